# baseline (speedup 1.0000x reference)
import jax
import jax.numpy as jnp
from jax import lax
from jax.experimental import pallas as pl
from jax.experimental.pallas import tpu as pltpu
from jax.experimental.pallas import tpu_sc as plsc

ROWS = 2097152
COLS = 4
BLOCK = 262144
NC = 2
NS = 16
NW = NC * NS
S = ROWS // NW
CROWS = 2048
NCHUNK = S // CROWS


def _body(in_hbm, zeros_hbm, out_hbm, sem):
    c = lax.axis_index("c")
    s = lax.axis_index("s")
    wid = s * NC + c
    in_off = wid * S
    k = wid // 4
    q = wid % 4
    out_off = k * (2 * BLOCK) + q * S
    zero_off = out_off + BLOCK
    copies = []
    for i in range(NCHUNK):
        r = i * CROWS
        copies.append(pltpu.async_copy(
            in_hbm.at[pl.ds(in_off + r, CROWS)],
            out_hbm.at[pl.ds(out_off + r, CROWS)], sem))
        copies.append(pltpu.async_copy(
            zeros_hbm.at[pl.ds(i * CROWS, CROWS)],
            out_hbm.at[pl.ds(zero_off + r, CROWS)], sem))
    for cp in copies:
        cp.wait()


def kernel(psi):
    zeros = jnp.zeros((S, COLS), jnp.float32)
    mesh = plsc.VectorSubcoreMesh(core_axis_name="c", subcore_axis_name="s")
    run = pl.kernel(
        _body,
        out_type=jax.ShapeDtypeStruct((2 * ROWS, COLS), jnp.float32),
        mesh=mesh,
        scratch_types=[pltpu.SemaphoreType.DMA],
    )
    return run(psi, zeros)


# speedup vs baseline: 32.4505x; 32.4505x over previous
import jax
import jax.numpy as jnp
from jax import lax
from jax.experimental import pallas as pl
from jax.experimental.pallas import tpu as pltpu
from jax.experimental.pallas import tpu_sc as plsc

ROWS = 2097152
COLS = 4
LANE = 128
IN_R = ROWS * COLS // LANE
OUT_R = 2 * IN_R
CHUNK_R = 8192
NC = 2
NS = 16
NW = NC * NS
S = IN_R // NW


def _body(in_hbm, zeros_hbm, out_hbm):
    c = lax.axis_index("c")
    s = lax.axis_index("s")
    wid = s * NC + c
    in_off = wid * S
    k = wid // 4
    q = wid % 4
    out_off = k * (2 * CHUNK_R) + q * S
    zero_off = out_off + CHUNK_R
    pltpu.sync_copy(in_hbm.at[pl.ds(in_off, S)], out_hbm.at[pl.ds(out_off, S)])
    pltpu.sync_copy(zeros_hbm.at[:], out_hbm.at[pl.ds(zero_off, S)])


def kernel(psi):
    flat = psi.reshape(ROWS // LANE, LANE, COLS).transpose(0, 2, 1)
    flat = flat.reshape(IN_R, LANE)
    zeros = jnp.zeros((S, LANE), jnp.float32)
    mesh = plsc.VectorSubcoreMesh(core_axis_name="c", subcore_axis_name="s")
    run = pl.kernel(
        _body,
        out_type=jax.ShapeDtypeStruct((OUT_R, LANE), jnp.float32),
        mesh=mesh,
    )
    out = run(flat, zeros)
    out = out.reshape(2 * ROWS // LANE, COLS, LANE).transpose(0, 2, 1)
    return out.reshape(2 * ROWS, COLS)
